# baseline (speedup 1.0000x reference)
import dataclasses
import functools

import jax
import jax.numpy as jnp
from jax import lax
from jax.experimental import pallas as pl
from jax.experimental.pallas import tpu as pltpu
from jax.experimental.pallas import tpu_sc as plsc

T = 2048
G = 16
NLANES = 16
NWORKERS = 32
CUNROLL = 8
SUP = 8
NSUPMAX = 5

_TIERS = ((2, (2 * SUP - 10) * T // (G - 1)),
          (3, (3 * SUP - 10) * T // (G - 1)),
          (4, (4 * SUP - 10) * T // (G - 1)),
          (5, None))


def _splat(vec, i):
    idx = jnp.full((NLANES, 1), i, dtype=jnp.int32)
    dn = lax.GatherDimensionNumbers(
        offset_dims=(), collapsed_slice_dims=(0,), start_index_map=(0,))
    return lax.gather(vec, idx, dn, slice_sizes=(1,),
                      mode=lax.GatherScatterMode.PROMISE_IN_BOUNDS)


def _make_sc_resample(B, Lmax, C):
    N = B * T
    mesh = plsc.VectorSubcoreMesh(core_axis_name="c", subcore_axis_name="s")
    rows_per_worker = N // NWORKERS
    chpw = rows_per_worker // G
    wpb = NWORKERS // B

    cp = pltpu.CompilerParams()
    if "needs_layout_passes" in pltpu.CompilerParams.__dataclass_fields__:
        cp = dataclasses.replace(cp, needs_layout_passes=False)

    @functools.partial(
        pl.kernel,
        mesh=mesh,
        compiler_params=cp,
        out_type=jax.ShapeDtypeStruct((N, C), jnp.float32),
        scratch_types=[
            pltpu.VMEM((1, NLANES), jnp.int32),
            pltpu.VMEM((NLANES,), jnp.int32),
            pltpu.VMEM((NLANES,), jnp.int32),
            pltpu.VMEM((NSUPMAX, SUP, C), jnp.float32),
            pltpu.VMEM((NSUPMAX, SUP, C), jnp.float32),
            pltpu.VMEM((G, C), jnp.float32),
            pltpu.VMEM((G, C), jnp.float32),
            pltpu.SemaphoreType.DMA,
            pltpu.SemaphoreType.DMA,
            pltpu.SemaphoreType.DMA,
            pltpu.SemaphoreType.DMA,
        ],
    )
    def sc_resample(x_hbm, l_hbm, out_hbm,
                    lv, gi_a, gi_b, win_a, win_b, o_a, o_b,
                    sf_a, sf_b, so_a, so_b):
        gi = (gi_a, gi_b)
        win = (win_a, win_b)
        ov = (o_a, o_b)
        sf = (sf_a, sf_b)
        so = (so_a, so_b)

        wid = lax.axis_index("s") * 2 + lax.axis_index("c")
        row0 = wid * rows_per_worker
        bb = wid // wpb
        sroff = bb * (Lmax // SUP)

        pltpu.sync_copy(l_hbm, lv)
        lvv = lv[0, :]
        Lvec = _splat(lvv, bb)
        Lscal = Lvec[0]
        Lfvec = Lvec.astype(jnp.float32)
        scale = Lfvec * (1.0 / float(T))
        lim0 = Lfvec - 1.0
        lim1 = Lvec - 1

        iota = lax.iota(jnp.int32, NLANES)
        iota_f = iota.astype(jnp.float32)

        def chunk_math(k):
            j0 = (wid % wpb) * rows_per_worker + k * G
            jv = j0.astype(jnp.float32) + iota_f
            src = (jv + 0.5) * scale - 0.5
            src = jnp.minimum(jnp.maximum(src, 0.0), lim0)
            i0 = src.astype(jnp.int32)
            i1 = jnp.minimum(i0 + 1, lim1)
            w = src - i0.astype(jnp.float32)
            return i0, i1, w

        def sup_start(i0):
            return jnp.minimum(i0[0] // SUP, Lmax // SUP - NSUPMAX)

        def tier_when(nsup, lmax_tier, fn):
            if lmax_tier is None:
                cond = Lscal > _TIERS[-2][1]
            elif nsup == _TIERS[0][0]:
                cond = Lscal <= lmax_tier
            else:
                lo = [l_ for n_, l_ in _TIERS if n_ < nsup][-1]
                cond = (Lscal > lo) & (Lscal <= lmax_tier)
            pl.when(cond)(fn)

        def fill_op(k, s, do_start):
            i0, _, _ = chunk_math(k)
            ss = sup_start(i0)
            if do_start:
                gi[s][...] = sroff + ss + iota
            for nsup, lmax_tier in _TIERS:
                def body(nsup=nsup):
                    cp_ = pltpu.make_async_copy(
                        x_hbm.at[gi[s].at[pl.ds(0, nsup)]],
                        win[s].at[pl.ds(0, nsup)], sf[s])
                    if do_start:
                        cp_.start()
                    else:
                        cp_.wait()
                tier_when(nsup, lmax_tier, body)

        def out_copy(k, s):
            return pltpu.make_async_copy(
                ov[s], out_hbm.at[pl.ds(row0 + k * G, G), :], so[s])

        @pl.loop(0, chpw, step=2)
        def _(k0):
            for slot in range(2):
                k = k0 + slot
                s, ns = slot, 1 - slot

                fill_op(k, s, True)
                fill_op(k, s, False)

                @pl.when(k >= 2)
                def _():
                    out_copy(k, s).wait()

                i0, i1, w = chunk_math(k)
                start = sup_start(i0) * SUP
                l0v = i0 - start
                l1v = i1 - start
                l0sup, l0sub = l0v // SUP, l0v % SUP
                l1sup, l1sub = l1v // SUP, l1v % SUP
                for r in range(G):
                    wspl = _splat(w, r)
                    a_sup = _splat(l0sup, r)
                    a_sub = _splat(l0sub, r)
                    b_sup = _splat(l1sup, r)
                    b_sub = _splat(l1sub, r)

                    @pl.loop(0, C, step=NLANES * CUNROLL)
                    def _(cc):
                        for u in range(CUNROLL):
                            lane = iota + (cc + u * NLANES)
                            a = plsc.load_gather(win[s], [a_sup, a_sub, lane])
                            b2 = plsc.load_gather(win[s], [b_sup, b_sub, lane])
                            ov[s][r, pl.ds(cc + u * NLANES, NLANES)] = (
                                a + wspl * (b2 - a))

                out_copy(k, s).start()

        out_copy(chpw - 2, 0).wait()
        out_copy(chpw - 1, 1).wait()

    return sc_resample


def kernel(x, lengths):
    B, Lmax, C = x.shape
    x3 = x.reshape(B * Lmax // SUP, SUP, C)
    lp = jnp.pad(lengths, (0, NLANES - B)).reshape(1, NLANES)
    out2 = _make_sc_resample(B, Lmax, C)(x3, lp)
    return out2.reshape(B, T, C)


# speedup vs baseline: 3.3878x; 3.3878x over previous
import dataclasses
import functools

import jax
import jax.numpy as jnp
from jax import lax
from jax.experimental import pallas as pl
from jax.experimental.pallas import tpu as pltpu
from jax.experimental.pallas import tpu_sc as plsc

T = 2048
G = 16
NLANES = 16
NWORKERS = 32
CUNROLL = 8


def _splat(vec, i):
    idx = jnp.full((NLANES, 1), i, dtype=jnp.int32)
    dn = lax.GatherDimensionNumbers(
        offset_dims=(), collapsed_slice_dims=(0,), start_index_map=(0,))
    return lax.gather(vec, idx, dn, slice_sizes=(1,),
                      mode=lax.GatherScatterMode.PROMISE_IN_BOUNDS)


def _make_sc_resample(B, Lmax, C):
    N = B * T
    mesh = plsc.VectorSubcoreMesh(core_axis_name="c", subcore_axis_name="s")
    rows_per_worker = N // NWORKERS
    chpw = rows_per_worker // G
    wpb = NWORKERS // B

    cp = pltpu.CompilerParams()
    if "needs_layout_passes" in pltpu.CompilerParams.__dataclass_fields__:
        cp = dataclasses.replace(cp, needs_layout_passes=False)

    @functools.partial(
        pl.kernel,
        mesh=mesh,
        compiler_params=cp,
        out_type=jax.ShapeDtypeStruct((N, C), jnp.float32),
        scratch_types=[
            pltpu.VMEM((1, NLANES), jnp.int32),
            pltpu.VMEM((2 * G, C), jnp.float32),
            pltpu.VMEM((2 * G, C), jnp.float32),
            pltpu.VMEM((G, C), jnp.float32),
            pltpu.VMEM((G, C), jnp.float32),
            pltpu.SemaphoreType.DMA,
            pltpu.SemaphoreType.DMA,
            pltpu.SemaphoreType.DMA,
            pltpu.SemaphoreType.DMA,
            pltpu.SemaphoreType.DMA,
            pltpu.SemaphoreType.DMA,
        ],
    )
    def sc_resample(x_hbm, l_hbm, out_hbm,
                    lv, win_a, win_b, o_a, o_b,
                    sg0_a, sg0_b, sg1_a, sg1_b, so_a, so_b):
        win = (win_a, win_b)
        ov = (o_a, o_b)
        sg0 = (sg0_a, sg0_b)
        sg1 = (sg1_a, sg1_b)
        so = (so_a, so_b)

        wid = lax.axis_index("s") * 2 + lax.axis_index("c")
        row0 = wid * rows_per_worker

        pltpu.sync_copy(l_hbm, lv)
        lvv = lv[0, :]

        iota = lax.iota(jnp.int32, NLANES)
        iota_f = iota.astype(jnp.float32)

        def chunk_math(bb, k):
            L = _splat(lvv, bb)
            Lf = L.astype(jnp.float32)
            scale = Lf * (1.0 / float(T))
            j0 = (wid % wpb) * rows_per_worker + k * G
            jv = j0.astype(jnp.float32) + iota_f
            src = (jv + 0.5) * scale - 0.5
            src = jnp.minimum(jnp.maximum(src, 0.0), Lf - 1.0)
            i0 = src.astype(jnp.int32)
            i1 = jnp.minimum(i0 + 1, L - 1)
            w = src - i0.astype(jnp.float32)
            return i0, i1, w

        def fire(bb, roff, k, s):
            i0, i1, _ = chunk_math(bb, k)
            pltpu.make_async_copy(x_hbm.at[roff + i0],
                                  win[s].at[pl.ds(0, G), :], sg0[s]).start()
            pltpu.make_async_copy(x_hbm.at[roff + i1],
                                  win[s].at[pl.ds(G, G), :], sg1[s]).start()

        def wait_fill(s):
            pltpu.make_async_copy(x_hbm.at[iota],
                                  win[s].at[pl.ds(0, G), :], sg0[s]).wait()
            pltpu.make_async_copy(x_hbm.at[iota],
                                  win[s].at[pl.ds(G, G), :], sg1[s]).wait()

        def out_copy(k, s):
            return pltpu.make_async_copy(
                ov[s], out_hbm.at[pl.ds(row0 + k * G, G), :], so[s])

        bb = wid // wpb
        roff = bb * Lmax

        fire(bb, roff, 0, 0)

        @pl.loop(0, chpw, step=2)
        def _(k0):
            for slot in range(2):
                k = k0 + slot
                s, ns = slot, 1 - slot

                @pl.when(k + 1 < chpw)
                def _():
                    fire(bb, roff, k + 1, ns)

                wait_fill(s)

                @pl.when(k >= 2)
                def _():
                    out_copy(k, s).wait()

                _, _, w = chunk_math(bb, k)
                for r in range(G):
                    wspl = _splat(w, r)

                    @pl.loop(0, C, step=NLANES * CUNROLL)
                    def _(cc):
                        for u in range(CUNROLL):
                            sl = pl.ds(cc + u * NLANES, NLANES)
                            a = win[s][r, sl]
                            b2 = win[s][G + r, sl]
                            ov[s][r, sl] = a + wspl * (b2 - a)

                out_copy(k, s).start()

        out_copy(chpw - 2, 0).wait()
        out_copy(chpw - 1, 1).wait()

    return sc_resample


def kernel(x, lengths):
    B, Lmax, C = x.shape
    x2 = x.reshape(B * Lmax, C)
    lp = jnp.pad(lengths, (0, NLANES - B)).reshape(1, NLANES)
    out2 = _make_sc_resample(B, Lmax, C)(x2, lp)
    return out2.reshape(B, T, C)


# speedup vs baseline: 3.4626x; 1.0221x over previous
import dataclasses
import functools

import jax
import jax.numpy as jnp
from jax import lax
from jax.experimental import pallas as pl
from jax.experimental.pallas import tpu as pltpu
from jax.experimental.pallas import tpu_sc as plsc

T = 2048
G = 16
NLANES = 16
NWORKERS = 32
CUNROLL = 8


def _splat(vec, i):
    idx = jnp.full((NLANES, 1), i, dtype=jnp.int32)
    dn = lax.GatherDimensionNumbers(
        offset_dims=(), collapsed_slice_dims=(0,), start_index_map=(0,))
    return lax.gather(vec, idx, dn, slice_sizes=(1,),
                      mode=lax.GatherScatterMode.PROMISE_IN_BOUNDS)


def _make_sc_resample(B, Lmax, C):
    N = B * T
    mesh = plsc.VectorSubcoreMesh(core_axis_name="c", subcore_axis_name="s")
    rows_per_worker = N // NWORKERS
    chpw = rows_per_worker // G
    wpb = NWORKERS // B

    cp = pltpu.CompilerParams()
    if "needs_layout_passes" in pltpu.CompilerParams.__dataclass_fields__:
        cp = dataclasses.replace(cp, needs_layout_passes=False)

    @functools.partial(
        pl.kernel,
        mesh=mesh,
        compiler_params=cp,
        out_type=jax.ShapeDtypeStruct((N, C), jnp.float32),
        scratch_types=[
            pltpu.VMEM((1, NLANES), jnp.int32),
            pltpu.VMEM((2 * G, C), jnp.float32),
            pltpu.VMEM((2 * G, C), jnp.float32),
            pltpu.VMEM((G, C), jnp.float32),
            pltpu.VMEM((G, C), jnp.float32),
            pltpu.SemaphoreType.DMA,
            pltpu.SemaphoreType.DMA,
            pltpu.SemaphoreType.DMA,
            pltpu.SemaphoreType.DMA,
            pltpu.SemaphoreType.DMA,
            pltpu.SemaphoreType.DMA,
        ],
    )
    def sc_resample(x_hbm, l_hbm, out_hbm,
                    lv, win_a, win_b, o_a, o_b,
                    sg0_a, sg0_b, sg1_a, sg1_b, so_a, so_b):
        win = (win_a, win_b)
        ov = (o_a, o_b)
        sg0 = (sg0_a, sg0_b)
        sg1 = (sg1_a, sg1_b)
        so = (so_a, so_b)

        wid = lax.axis_index("s") * 2 + lax.axis_index("c")
        row0 = wid * rows_per_worker

        pltpu.sync_copy(l_hbm, lv)
        lvv = lv[0, :]

        iota = lax.iota(jnp.int32, NLANES)
        iota_f = iota.astype(jnp.float32)

        def chunk_math(bb, k):
            L = _splat(lvv, bb)
            Lf = L.astype(jnp.float32)
            scale = Lf * (1.0 / float(T))
            j0 = (wid % wpb) * rows_per_worker + k * G
            jv = j0.astype(jnp.float32) + iota_f
            src = (jv + 0.5) * scale - 0.5
            src = jnp.minimum(jnp.maximum(src, 0.0), Lf - 1.0)
            i0 = src.astype(jnp.int32)
            i1 = jnp.minimum(i0 + 1, L - 1)
            w = src - i0.astype(jnp.float32)
            return i0, i1, w

        def fire(bb, roff, k, s):
            i0, i1, _ = chunk_math(bb, k)
            pltpu.make_async_copy(x_hbm.at[roff + i0],
                                  win[s].at[pl.ds(0, G), :], sg0[s]).start()
            pltpu.make_async_copy(x_hbm.at[roff + i1],
                                  win[s].at[pl.ds(G, G), :], sg0[s]).start()

        def wait_fill(s):
            pltpu.make_async_copy(x_hbm.at[iota],
                                  win[s].at[pl.ds(0, G), :], sg0[s]).wait()
            pltpu.make_async_copy(x_hbm.at[iota],
                                  win[s].at[pl.ds(G, G), :], sg0[s]).wait()

        def out_copy(k, s):
            return pltpu.make_async_copy(
                ov[s], out_hbm.at[pl.ds(row0 + k * G, G), :], so[s])

        bb = wid // wpb
        roff = bb * Lmax

        fire(bb, roff, 0, 0)

        @pl.loop(0, chpw, step=2)
        def _(k0):
            for slot in range(2):
                k = k0 + slot
                s, ns = slot, 1 - slot

                @pl.when(k + 1 < chpw)
                def _():
                    fire(bb, roff, k + 1, ns)

                wait_fill(s)

                @pl.when(k >= 2)
                def _():
                    out_copy(k, s).wait()

                _, _, w = chunk_math(bb, k)
                for r in range(G):
                    wspl = _splat(w, r)

                    @pl.loop(0, C, step=NLANES * CUNROLL)
                    def _(cc):
                        for u in range(CUNROLL):
                            sl = pl.ds(cc + u * NLANES, NLANES)
                            a = win[s][r, sl]
                            b2 = win[s][G + r, sl]
                            ov[s][r, sl] = a + wspl * (b2 - a)

                out_copy(k, s).start()

        out_copy(chpw - 2, 0).wait()
        out_copy(chpw - 1, 1).wait()

    return sc_resample


def kernel(x, lengths):
    B, Lmax, C = x.shape
    x2 = x.reshape(B * Lmax, C)
    lp = jnp.pad(lengths, (0, NLANES - B)).reshape(1, NLANES)
    out2 = _make_sc_resample(B, Lmax, C)(x2, lp)
    return out2.reshape(B, T, C)
